# baseline (device time: 59175 ns/iter reference)
import jax
import jax.numpy as jnp
from jax import lax
from jax.experimental import pallas as pl
from jax.experimental.pallas import tpu as pltpu

N_DEV = 16


def kernel(x, Win0, Wout0, Win1, Wout1, Win2, Wout2):
    b_per, d = x.shape
    n_total = N_DEV * b_per

    def body(x_ref, win0_ref, wout0_ref, win1_ref, wout1_ref, win2_ref,
             wout2_ref, out_ref, xfull_ref, rs_ref, psend_ref,
             send_sems, ag_recv_sems, rs_recv_sems):
        my = lax.axis_index("i")

        barrier_sem = pltpu.get_barrier_semaphore()
        for off in range(1, N_DEV):
            t = (my + off) % N_DEV
            pl.semaphore_signal(
                barrier_sem, inc=1, device_id=(t,),
                device_id_type=pl.DeviceIdType.MESH)
        pl.semaphore_wait(barrier_sem, N_DEV - 1)

        def send_all(src_slot_fn, dst_ref, recv_sems):
            rdmas = []
            for off in range(1, N_DEV):
                t = (my + off) % N_DEV
                rdma = pltpu.make_async_remote_copy(
                    src_ref=src_slot_fn(t),
                    dst_ref=dst_ref.at[pl.ds(my, 1)],
                    send_sem=send_sems.at[off],
                    recv_sem=recv_sems.at[my],
                    device_id=(t,),
                    device_id_type=pl.DeviceIdType.MESH)
                rdma.start()
                rdmas.append(rdma)
            return rdmas

        def wait_recvs(dst_ref, recv_sems):
            for off in range(1, N_DEV):
                s = (my + off) % N_DEV
                rdma = pltpu.make_async_remote_copy(
                    src_ref=dst_ref.at[pl.ds(s, 1)],
                    dst_ref=dst_ref.at[pl.ds(s, 1)],
                    send_sem=send_sems.at[off],
                    recv_sem=recv_sems.at[s],
                    device_id=(s,),
                    device_id_type=pl.DeviceIdType.MESH)
                rdma.wait_recv()

        def wait_sends(rdmas):
            for rdma in rdmas:
                rdma.wait_send()

        def layer(win_ref, wout_ref):
            xf = xfull_ref[...].reshape(n_total, d)
            h = jnp.dot(xf, win_ref[...].astype(jnp.bfloat16),
                        preferred_element_type=jnp.float32)
            h = jnp.maximum(h, 0.0).astype(jnp.bfloat16)
            return jnp.dot(h, wout_ref[...].astype(jnp.bfloat16),
                           preferred_element_type=jnp.float32)

        def rs_round(p):
            psend_ref[...] = p.reshape(N_DEV, b_per, d).astype(jnp.bfloat16)
            rs_ref[pl.ds(my, 1), :, :] = psend_ref[pl.ds(my, 1), :, :]
            rdmas = send_all(lambda t: psend_ref.at[pl.ds(t, 1)],
                             rs_ref, rs_recv_sems)
            wait_recvs(rs_ref, rs_recv_sems)
            wait_sends(rdmas)
            return jnp.sum(rs_ref[...].astype(jnp.float32), axis=0)

        def ag_round(chunk_bf16):
            xfull_ref[pl.ds(my, 1), :, :] = chunk_bf16
            rdmas = send_all(lambda t: xfull_ref.at[pl.ds(my, 1)],
                             xfull_ref, ag_recv_sems)
            wait_recvs(xfull_ref, ag_recv_sems)
            wait_sends(rdmas)

        ag_round(x_ref[...].astype(jnp.bfloat16)[None])

        red0 = rs_round(layer(win0_ref, wout0_ref))
        ag_round(red0.astype(jnp.bfloat16)[None])

        red1 = rs_round(layer(win1_ref, wout1_ref))
        ag_round(red1.astype(jnp.bfloat16)[None])

        red2 = rs_round(layer(win2_ref, wout2_ref))
        out_ref[...] = red2

    return pl.pallas_call(
        body,
        out_shape=jax.ShapeDtypeStruct((b_per, d), jnp.float32),
        in_specs=[pl.BlockSpec(memory_space=pltpu.VMEM)] * 7,
        out_specs=pl.BlockSpec(memory_space=pltpu.VMEM),
        scratch_shapes=[
            pltpu.VMEM((N_DEV, b_per, d), jnp.bfloat16),
            pltpu.VMEM((N_DEV, b_per, d), jnp.bfloat16),
            pltpu.VMEM((N_DEV, b_per, d), jnp.bfloat16),
            pltpu.SemaphoreType.DMA((N_DEV,)),
            pltpu.SemaphoreType.DMA((N_DEV,)),
            pltpu.SemaphoreType.DMA((N_DEV,)),
        ],
        compiler_params=pltpu.CompilerParams(collective_id=0),
    )(x, Win0, Wout0, Win1, Wout1, Win2, Wout2)


# device time: 51384 ns/iter; 1.1516x vs baseline; 1.1516x over previous
import jax
import jax.numpy as jnp
from jax import lax
from jax.experimental import pallas as pl
from jax.experimental.pallas import tpu as pltpu

N_DEV = 16
GROUPS = 2


def kernel(x, Win0, Wout0, Win1, Wout1, Win2, Wout2):
    b_per, d = x.shape
    b_g = b_per // GROUPS

    def body(x_ref, win0_ref, wout0_ref, win1_ref, wout1_ref, win2_ref,
             wout2_ref, out_ref, xfull_ref, rs_ref, psend_ref,
             send_sems, ag_recv_sems, rs_recv_sems):
        my = lax.axis_index("i")

        def rows(g):
            return pl.ds(g * b_g, b_g)

        barrier_sem = pltpu.get_barrier_semaphore()
        for off in range(1, N_DEV):
            t = (my + off) % N_DEV
            pl.semaphore_signal(
                barrier_sem, inc=1, device_id=(t,),
                device_id_type=pl.DeviceIdType.MESH)
        pl.semaphore_wait(barrier_sem, N_DEV - 1)

        pending = {g: [] for g in range(GROUPS)}

        def start_sends(g, src_slot_fn, dst_ref, recv_sems):
            for r in pending[g]:
                r.wait_send()
            rds = []
            for off in range(1, N_DEV):
                t = (my + off) % N_DEV
                rdma = pltpu.make_async_remote_copy(
                    src_ref=src_slot_fn(t, g),
                    dst_ref=dst_ref.at[pl.ds(my, 1), rows(g), :],
                    send_sem=send_sems.at[g, off],
                    recv_sem=recv_sems.at[g, my],
                    device_id=(t,),
                    device_id_type=pl.DeviceIdType.MESH)
                rdma.start()
                rds.append(rdma)
            pending[g] = rds

        def wait_recvs(g, dst_ref, recv_sems):
            for off in range(1, N_DEV):
                s = (my + off) % N_DEV
                rdma = pltpu.make_async_remote_copy(
                    src_ref=dst_ref.at[pl.ds(s, 1), rows(g), :],
                    dst_ref=dst_ref.at[pl.ds(s, 1), rows(g), :],
                    send_sem=send_sems.at[g, off],
                    recv_sem=recv_sems.at[g, s],
                    device_id=(s,),
                    device_id_type=pl.DeviceIdType.MESH)
                rdma.wait_recv()

        def ag_src(t, g):
            return xfull_ref.at[pl.ds(my, 1), rows(g), :]

        def rs_src(t, g):
            return psend_ref.at[pl.ds(t, 1), rows(g), :]

        def layer(g, win_ref, wout_ref):
            xf = xfull_ref[:, rows(g), :].reshape(N_DEV * b_g, d)
            h = jnp.dot(xf, win_ref[...].astype(jnp.bfloat16),
                        preferred_element_type=jnp.float32)
            h = jnp.maximum(h, 0.0).astype(jnp.bfloat16)
            p = jnp.dot(h, wout_ref[...].astype(jnp.bfloat16),
                        preferred_element_type=jnp.float32)
            psend_ref[:, rows(g), :] = p.reshape(N_DEV, b_g, d).astype(
                jnp.bfloat16)
            rs_ref[pl.ds(my, 1), rows(g), :] = psend_ref[pl.ds(my, 1),
                                                         rows(g), :]
            start_sends(g, rs_src, rs_ref, rs_recv_sems)

        def reduce_and_bcast(g):
            wait_recvs(g, rs_ref, rs_recv_sems)
            red = jnp.sum(rs_ref[:, rows(g), :].astype(jnp.float32), axis=0)
            xfull_ref[pl.ds(my, 1), rows(g), :] = red.astype(jnp.bfloat16)[None]
            start_sends(g, ag_src, xfull_ref, ag_recv_sems)
            return red

        xfull_ref[pl.ds(my, 1), :, :] = x_ref[...].astype(jnp.bfloat16)[None]
        for g in range(GROUPS):
            start_sends(g, ag_src, xfull_ref, ag_recv_sems)

        for li, (win_ref, wout_ref) in enumerate(
                [(win0_ref, wout0_ref), (win1_ref, wout1_ref),
                 (win2_ref, wout2_ref)]):
            for g in range(GROUPS):
                wait_recvs(g, xfull_ref, ag_recv_sems)
                layer(g, win_ref, wout_ref)
            if li < 2:
                for g in range(GROUPS):
                    reduce_and_bcast(g)
            else:
                for g in range(GROUPS):
                    wait_recvs(g, rs_ref, rs_recv_sems)
                    out_ref[rows(g), :] = jnp.sum(
                        rs_ref[:, rows(g), :].astype(jnp.float32), axis=0)

        for g in range(GROUPS):
            for r in pending[g]:
                r.wait_send()

    return pl.pallas_call(
        body,
        out_shape=jax.ShapeDtypeStruct((b_per, d), jnp.float32),
        in_specs=[pl.BlockSpec(memory_space=pltpu.VMEM)] * 7,
        out_specs=pl.BlockSpec(memory_space=pltpu.VMEM),
        scratch_shapes=[
            pltpu.VMEM((N_DEV, b_per, d), jnp.bfloat16),
            pltpu.VMEM((N_DEV, b_per, d), jnp.bfloat16),
            pltpu.VMEM((N_DEV, b_per, d), jnp.bfloat16),
            pltpu.SemaphoreType.DMA((GROUPS, N_DEV)),
            pltpu.SemaphoreType.DMA((GROUPS, N_DEV)),
            pltpu.SemaphoreType.DMA((GROUPS, N_DEV)),
        ],
        compiler_params=pltpu.CompilerParams(collective_id=0),
    )(x, Win0, Wout0, Win1, Wout1, Win2, Wout2)
